# Initial kernel scaffold; baseline (speedup 1.0000x reference)
import jax
import jax.numpy as jnp
from jax.experimental import pallas as pl


def kernel(obj_vecs, edges, g1, b1, W1, c1, g2, b2, W2, c2):
    raise NotImplementedError("write your pallas kernel here")


# speedup vs baseline: 5133.0467x; 5133.0467x over previous
import functools

import jax
import jax.numpy as jnp
from jax import lax
from jax.experimental import pallas as pl
from jax.experimental.pallas import tpu as pltpu
from jax.experimental.pallas import tpu_sc as plsc


def _pair_hist_sc(s_idx, o_idx, num_obj):
    V, E = s_idx.shape
    OO = num_obj * num_obj
    mesh = plsc.VectorSubcoreMesh(core_axis_name="c", subcore_axis_name="s")

    @functools.partial(
        pl.kernel,
        mesh=mesh,
        out_type=jax.ShapeDtypeStruct((V, OO), jnp.float32),
        scratch_types=[
            pltpu.VMEM((E,), jnp.int32),
            pltpu.VMEM((E,), jnp.int32),
            pltpu.VMEM((OO + 16,), jnp.float32),
        ],
    )
    def hist(s_hbm, o_hbm, out_hbm, s_v, o_v, acc_v):
        wid = lax.axis_index("s") * 2 + lax.axis_index("c")

        def zero_body(i, carry):
            acc_v[pl.ds(i * 16, 16)] = jnp.zeros((16,), jnp.float32)
            return carry

        lax.fori_loop(0, OO // 16 + 1, zero_body, 0)

        pltpu.sync_copy(s_hbm.at[wid], s_v)
        pltpu.sync_copy(o_hbm.at[wid], o_v)

        one0 = jnp.where(lax.iota(jnp.int32, 16) == 0, 1.0, 0.0)

        def body(g, carry):
            codes = s_v[pl.ds(g * 16, 16)] * num_obj + o_v[pl.ds(g * 16, 16)]
            for l in range(16):
                c = codes[l]
                acc_v[pl.ds(c, 16)] = acc_v[pl.ds(c, 16)] + one0
            return carry

        lax.fori_loop(0, E // 16, body, 0)

        pltpu.sync_copy(acc_v.at[pl.ds(0, OO)], out_hbm.at[wid])

    return hist(s_idx, o_idx)


def _stats1_body(x_ref, c_ref, stat_ref):
    v = pl.program_id(0)
    x = x_ref[0]
    C = c_ref[0]
    x2 = x * x
    cnt_s = jnp.sum(C, axis=1, keepdims=True)
    s_src = jnp.sum(cnt_s * x, axis=0, keepdims=True)
    q_src = jnp.sum(cnt_s * x2, axis=0, keepdims=True)
    Cx = jnp.dot(C, x, preferred_element_type=jnp.float32)
    Cx2 = jnp.dot(C, x2, preferred_element_type=jnp.float32)
    s_dst = jnp.sum(Cx, axis=0, keepdims=True)
    q_dst = jnp.sum(Cx2, axis=0, keepdims=True)
    block = jnp.concatenate(
        [s_src, q_src, s_dst, q_dst, jnp.zeros((4, s_src.shape[1]), jnp.float32)],
        axis=0,
    )

    @pl.when(v == 0)
    def _():
        stat_ref[...] = block

    @pl.when(v > 0)
    def _():
        stat_ref[...] += block


def _pool_body(x_ref, c_ref, stat_ref, par_ref, w1_ref, pooled_ref, st2_ref, *, n1, d):
    v = pl.program_id(0)
    x = x_ref[0]
    C = c_ref[0]
    inv_n1 = 1.0 / n1
    mean_s = stat_ref[0:1] * inv_n1
    var_s = stat_ref[1:2] * inv_n1 - mean_s * mean_s
    mean_d = stat_ref[2:3] * inv_n1
    var_d = stat_ref[3:4] * inv_n1 - mean_d * mean_d
    scale_s = par_ref[0:1] * lax.rsqrt(var_s + 1e-5)
    scale_d = par_ref[1:2] * lax.rsqrt(var_d + 1e-5)
    shift_s = par_ref[2:3] - mean_s * scale_s
    shift_d = par_ref[3:4] - mean_d * scale_d
    c1 = par_ref[4:5]

    a_in = jnp.maximum(x * scale_s + shift_s, 0.0)
    b_in = jnp.maximum(x * scale_d + shift_d, 0.0)
    A = jnp.dot(a_in, w1_ref[0:d], preferred_element_type=jnp.float32)
    B = jnp.dot(b_in, w1_ref[d:2 * d], preferred_element_type=jnp.float32)

    cnt = jnp.sum(C, axis=1, keepdims=True)
    num = cnt * (A + c1) + jnp.dot(C, B, preferred_element_type=jnp.float32)
    pooled = num / jnp.maximum(cnt, 1.0)
    pooled_ref[0] = pooled

    s2 = jnp.sum(pooled, axis=0, keepdims=True)
    q2 = jnp.sum(pooled * pooled, axis=0, keepdims=True)
    block = jnp.concatenate(
        [s2, q2, jnp.zeros((6, s2.shape[1]), jnp.float32)], axis=0)

    @pl.when(v == 0)
    def _():
        st2_ref[...] = block

    @pl.when(v > 0)
    def _():
        st2_ref[...] += block


def _out_body(p_ref, st2_ref, par_ref, w2_ref, out_ref, *, n2):
    p = p_ref[0]
    inv_n2 = 1.0 / n2
    mean2 = st2_ref[0:1] * inv_n2
    var2 = st2_ref[1:2] * inv_n2 - mean2 * mean2
    scale2 = par_ref[5:6] * lax.rsqrt(var2 + 1e-5)
    shift2 = par_ref[6:7] - mean2 * scale2
    y = jnp.maximum(p * scale2 + shift2, 0.0)
    out_ref[0] = jnp.dot(y, w2_ref[...], preferred_element_type=jnp.float32) + par_ref[7:8]


def _tc_pipeline(obj_vecs, C, params8, W1, W2, n1, interpret=False):
    V, O, D = obj_vecs.shape
    full = lambda shape: pl.BlockSpec(shape, lambda v: (0,) * len(shape))
    per_v = lambda shape: pl.BlockSpec((1,) + shape, lambda v: (v,) + (0,) * len(shape))

    stats1 = pl.pallas_call(
        _stats1_body,
        grid=(V,),
        in_specs=[per_v((O, D)), per_v((O, O))],
        out_specs=full((8, D)),
        out_shape=jax.ShapeDtypeStruct((8, D), jnp.float32),
        interpret=interpret,
    )(obj_vecs, C)

    pooled, stats2 = pl.pallas_call(
        functools.partial(_pool_body, n1=float(n1), d=D),
        grid=(V,),
        in_specs=[per_v((O, D)), per_v((O, O)), full((8, D)), full((8, D)),
                  full((2 * D, D))],
        out_specs=[per_v((O, D)), full((8, D))],
        out_shape=[jax.ShapeDtypeStruct((V, O, D), jnp.float32),
                   jax.ShapeDtypeStruct((8, D), jnp.float32)],
        interpret=interpret,
    )(obj_vecs, C, stats1, params8, W1)

    out = pl.pallas_call(
        functools.partial(_out_body, n2=float(V * O)),
        grid=(V,),
        in_specs=[per_v((O, D)), full((8, D)), full((8, D)), full((D, D))],
        out_specs=per_v((O, D)),
        out_shape=jax.ShapeDtypeStruct((V, O, D), jnp.float32),
        interpret=interpret,
    )(pooled, stats2, params8, W2)
    return out


def kernel(obj_vecs, edges, g1, b1, W1, c1, g2, b2, W2, c2):
    V, O, D = obj_vecs.shape
    E = edges.shape[1]
    s_idx = edges[:, :, 0]
    o_idx = edges[:, :, 2]
    params8 = jnp.stack([g1[:D], g1[D:], b1[:D], b1[D:], c1, g2, b2, c2], axis=0)

    C = _pair_hist_sc(s_idx, o_idx, O).reshape(V, O, O)

    return _tc_pipeline(obj_vecs, C, params8, W1, W2, V * E)


# speedup vs baseline: 5524.7735x; 1.0763x over previous
import functools

import jax
import jax.numpy as jnp
from jax import lax
from jax.experimental import pallas as pl
from jax.experimental.pallas import tpu as pltpu
from jax.experimental.pallas import tpu_sc as plsc


def _pair_hist_sc(s_idx, o_idx, num_obj):
    V, E = s_idx.shape
    OO = num_obj * num_obj
    mesh = plsc.VectorSubcoreMesh(core_axis_name="c", subcore_axis_name="s")

    @functools.partial(
        pl.kernel,
        mesh=mesh,
        out_type=jax.ShapeDtypeStruct((V, OO), jnp.float32),
        scratch_types=[
            pltpu.VMEM((E,), jnp.int32),
            pltpu.VMEM((E,), jnp.int32),
            pltpu.VMEM((OO + 16,), jnp.float32),
        ],
    )
    def hist(s_hbm, o_hbm, out_hbm, s_v, o_v, acc_v):
        wid = lax.axis_index("s") * 2 + lax.axis_index("c")

        def zero_body(i, carry):
            acc_v[pl.ds(i * 16, 16)] = jnp.zeros((16,), jnp.float32)
            return carry

        lax.fori_loop(0, OO // 16 + 1, zero_body, 0)

        pltpu.sync_copy(s_hbm.at[wid], s_v)
        pltpu.sync_copy(o_hbm.at[wid], o_v)

        one0 = jnp.where(lax.iota(jnp.int32, 16) == 0, 1.0, 0.0)

        def body(g, carry):
            codes = s_v[pl.ds(g * 16, 16)] * num_obj + o_v[pl.ds(g * 16, 16)]
            for l in range(16):
                c = codes[l]
                acc_v[pl.ds(c, 16)] = acc_v[pl.ds(c, 16)] + one0
            return carry

        lax.fori_loop(0, E // 16, body, 0)

        pltpu.sync_copy(acc_v.at[pl.ds(0, OO)], out_hbm.at[wid])

    return hist(s_idx, o_idx)


def _stats1_body(x_ref, c_ref, stat_ref):
    v = pl.program_id(0)
    x = x_ref[0]
    C = c_ref[0]
    x2 = x * x
    cnt_s = jnp.sum(C, axis=1, keepdims=True)
    s_src = jnp.sum(cnt_s * x, axis=0, keepdims=True)
    q_src = jnp.sum(cnt_s * x2, axis=0, keepdims=True)
    cnt_o = jnp.sum(C, axis=0, keepdims=True)
    s_dst = jnp.dot(cnt_o, x, preferred_element_type=jnp.float32)
    q_dst = jnp.dot(cnt_o, x2, preferred_element_type=jnp.float32)
    block = jnp.concatenate(
        [s_src, q_src, s_dst, q_dst, jnp.zeros((4, s_src.shape[1]), jnp.float32)],
        axis=0,
    )

    @pl.when(v == 0)
    def _():
        stat_ref[...] = block

    @pl.when(v > 0)
    def _():
        stat_ref[...] += block


def _pool_body(x_ref, c_ref, stat_ref, par_ref, w1_ref, pooled_ref, st2_ref, *, n1, d):
    v = pl.program_id(0)
    x = x_ref[0]
    C = c_ref[0]
    inv_n1 = 1.0 / n1
    mean_s = stat_ref[0:1] * inv_n1
    var_s = stat_ref[1:2] * inv_n1 - mean_s * mean_s
    mean_d = stat_ref[2:3] * inv_n1
    var_d = stat_ref[3:4] * inv_n1 - mean_d * mean_d
    scale_s = par_ref[0:1] * lax.rsqrt(var_s + 1e-5)
    scale_d = par_ref[1:2] * lax.rsqrt(var_d + 1e-5)
    shift_s = par_ref[2:3] - mean_s * scale_s
    shift_d = par_ref[3:4] - mean_d * scale_d
    c1 = par_ref[4:5]

    a_in = jnp.maximum(x * scale_s + shift_s, 0.0)
    b_in = jnp.maximum(x * scale_d + shift_d, 0.0)
    A = jnp.dot(a_in, w1_ref[0:d], preferred_element_type=jnp.float32)
    B = jnp.dot(b_in, w1_ref[d:2 * d], preferred_element_type=jnp.float32)

    cnt = jnp.sum(C, axis=1, keepdims=True)
    num = cnt * (A + c1) + jnp.dot(C, B, preferred_element_type=jnp.float32)
    pooled = num / jnp.maximum(cnt, 1.0)
    pooled_ref[0] = pooled

    s2 = jnp.sum(pooled, axis=0, keepdims=True)
    q2 = jnp.sum(pooled * pooled, axis=0, keepdims=True)
    block = jnp.concatenate(
        [s2, q2, jnp.zeros((6, s2.shape[1]), jnp.float32)], axis=0)

    @pl.when(v == 0)
    def _():
        st2_ref[...] = block

    @pl.when(v > 0)
    def _():
        st2_ref[...] += block


def _out_body(p_ref, st2_ref, par_ref, w2_ref, out_ref, *, n2):
    g, o, d = p_ref.shape
    p = p_ref[...].reshape(g * o, d)
    inv_n2 = 1.0 / n2
    mean2 = st2_ref[0:1] * inv_n2
    var2 = st2_ref[1:2] * inv_n2 - mean2 * mean2
    scale2 = par_ref[5:6] * lax.rsqrt(var2 + 1e-5)
    shift2 = par_ref[6:7] - mean2 * scale2
    y = jnp.maximum(p * scale2 + shift2, 0.0)
    out = jnp.dot(y, w2_ref[...], preferred_element_type=jnp.float32) + par_ref[7:8]
    out_ref[...] = out.reshape(g, o, d)


def _tc_pipeline(obj_vecs, C, params8, W1, W2, n1, interpret=False):
    V, O, D = obj_vecs.shape
    full = lambda shape: pl.BlockSpec(shape, lambda v: (0,) * len(shape))
    per_v = lambda shape: pl.BlockSpec((1,) + shape, lambda v: (v,) + (0,) * len(shape))

    stats1 = pl.pallas_call(
        _stats1_body,
        grid=(V,),
        in_specs=[per_v((O, D)), per_v((O, O))],
        out_specs=full((8, D)),
        out_shape=jax.ShapeDtypeStruct((8, D), jnp.float32),
        interpret=interpret,
    )(obj_vecs, C)

    pooled, stats2 = pl.pallas_call(
        functools.partial(_pool_body, n1=float(n1), d=D),
        grid=(V,),
        in_specs=[per_v((O, D)), per_v((O, O)), full((8, D)), full((8, D)),
                  full((2 * D, D))],
        out_specs=[per_v((O, D)), full((8, D))],
        out_shape=[jax.ShapeDtypeStruct((V, O, D), jnp.float32),
                   jax.ShapeDtypeStruct((8, D), jnp.float32)],
        interpret=interpret,
    )(obj_vecs, C, stats1, params8, W1)

    G3 = 4
    per_g = pl.BlockSpec((G3, O, D), lambda v: (v, 0, 0))
    out = pl.pallas_call(
        functools.partial(_out_body, n2=float(V * O)),
        grid=(V // G3,),
        in_specs=[per_g, full((8, D)), full((8, D)), full((D, D))],
        out_specs=per_g,
        out_shape=jax.ShapeDtypeStruct((V, O, D), jnp.float32),
        interpret=interpret,
    )(pooled, stats2, params8, W2)
    return out


def kernel(obj_vecs, edges, g1, b1, W1, c1, g2, b2, W2, c2):
    V, O, D = obj_vecs.shape
    E = edges.shape[1]
    s_idx = edges[:, :, 0]
    o_idx = edges[:, :, 2]
    params8 = jnp.stack([g1[:D], g1[D:], b1[:D], b1[D:], c1, g2, b2, c2], axis=0)

    C = _pair_hist_sc(s_idx, o_idx, O).reshape(V, O, O)

    return _tc_pipeline(obj_vecs, C, params8, W1, W2, V * E)
